# baseline (speedup 1.0000x reference)
import functools

import jax
import jax.numpy as jnp
from jax import lax
from jax.experimental import pallas as pl
from jax.experimental.pallas import tpu as pltpu
from jax.experimental.pallas import tpu_sc as plsc

NC = 2
NS = 16
NW = NC * NS
L = 16

IDX_CHUNK = 128
LINE = 128


def _mf_kernel(b_per_w, n_chunks, n_feats, n_elines, n_blines):
    mesh = plsc.VectorSubcoreMesh(core_axis_name="c", subcore_axis_name="s")
    B = b_per_w * NW
    rows_per_line = LINE // n_feats

    @functools.partial(
        pl.kernel,
        mesh=mesh,
        compiler_params=pltpu.CompilerParams(needs_layout_passes=False),
        out_type=jax.ShapeDtypeStruct((B,), jnp.float32),
        scratch_types=[
            pltpu.VMEM((b_per_w,), jnp.int32),
            pltpu.VMEM((b_per_w,), jnp.int32),
            pltpu.VMEM((b_per_w,), jnp.int32),
            pltpu.VMEM((b_per_w,), jnp.int32),
            pltpu.VMEM((b_per_w,), jnp.int32),
            pltpu.VMEM((b_per_w,), jnp.int32),
            pltpu.VMEM((IDX_CHUNK, LINE), jnp.float32),
            pltpu.VMEM((IDX_CHUNK, LINE), jnp.float32),
            pltpu.VMEM((IDX_CHUNK, LINE), jnp.float32),
            pltpu.VMEM((IDX_CHUNK, LINE), jnp.float32),
            pltpu.VMEM((b_per_w,), jnp.float32),
            pltpu.SemaphoreType.DMA,
        ],
    )
    def k(uraw_hbm, iraw_hbm, uel_hbm, iel_hbm, ubl_hbm, ibl_hbm,
          ue_hbm, ie_hbm, ub_hbm, ib_hbm, out_hbm,
          uraw_v, iraw_v, uel_v, iel_v, ubl_v, ibl_v,
          ue_v, ie_v, ub_v, ib_v, out_v, sem):
        wid = lax.axis_index("s") * NC + lax.axis_index("c")
        base = wid * b_per_w
        wsl = pl.ds(base, b_per_w)

        pltpu.sync_copy(uraw_hbm.at[wsl], uraw_v)
        pltpu.sync_copy(iraw_hbm.at[wsl], iraw_v)
        pltpu.sync_copy(uel_hbm.at[wsl], uel_v)
        pltpu.sync_copy(iel_hbm.at[wsl], iel_v)
        pltpu.sync_copy(ubl_hbm.at[wsl], ubl_v)
        pltpu.sync_copy(ibl_hbm.at[wsl], ibl_v)

        for j in range(n_chunks):
            csl = pl.ds(j * IDX_CHUNK, IDX_CHUNK)
            cps = [
                pltpu.async_copy(ue_hbm.at[uel_v.at[csl]], ue_v, sem),
                pltpu.async_copy(ie_hbm.at[iel_v.at[csl]], ie_v, sem),
                pltpu.async_copy(ub_hbm.at[ubl_v.at[csl]], ub_v, sem),
                pltpu.async_copy(ib_hbm.at[ibl_v.at[csl]], ib_v, sem),
            ]
            for c in cps:
                c.wait()

            def body(g, _, _j=j):
                loc = g * L
                glob = _j * IDX_CHUNK + loc
                rows = loc + lax.broadcasted_iota(jnp.int32, (L,), 0)
                uraw = uraw_v[pl.ds(glob, L)]
                iraw = iraw_v[pl.ds(glob, L)]
                ucol = (uraw & (rows_per_line - 1)) * n_feats
                icol = (iraw & (rows_per_line - 1)) * n_feats
                acc = (plsc.load_gather(ub_v, [rows, uraw & (LINE - 1)])
                       + plsc.load_gather(ib_v, [rows, iraw & (LINE - 1)]))
                for f in range(n_feats):
                    acc = acc + (plsc.load_gather(ue_v, [rows, ucol + f])
                                 * plsc.load_gather(ie_v, [rows, icol + f]))
                out_v[pl.ds(glob, L)] = acc
                return _

            lax.fori_loop(0, IDX_CHUNK // L, body, None)

        pltpu.sync_copy(out_v, out_hbm.at[wsl])

    return k


def kernel(user, item, u_bias, i_bias, u_embed, i_embed):
    B = user.shape[0]
    n_rows, n_feats = u_embed.shape
    rows_per_line = LINE // n_feats
    b_per_w = B // NW
    n_chunks = b_per_w // IDX_CHUNK

    user = user.astype(jnp.int32)
    item = item.astype(jnp.int32)

    ue = u_embed.reshape(n_rows // rows_per_line, LINE)
    ie = i_embed.reshape(n_rows // rows_per_line, LINE)

    n_blines = -(-n_rows // LINE)
    pad = n_blines * LINE - n_rows
    ub = jnp.pad(u_bias.reshape(-1), (0, pad)).reshape(n_blines, LINE)
    ib = jnp.pad(i_bias.reshape(-1), (0, pad)).reshape(n_blines, LINE)

    uel = user // rows_per_line
    iel = item // rows_per_line
    ubl = user // LINE
    ibl = item // LINE

    k = _mf_kernel(b_per_w, n_chunks, n_feats, n_rows // rows_per_line,
                   n_blines)
    return k(user, item, uel, iel, ubl, ibl, ue, ie, ub, ib)


# speedup vs baseline: 3.3848x; 3.3848x over previous
import functools

import jax
import jax.numpy as jnp
from jax import lax
from jax.experimental import pallas as pl
from jax.experimental.pallas import tpu as pltpu
from jax.experimental.pallas import tpu_sc as plsc

NC = 2
NS = 16
NW = NC * NS
L = 16

CHUNK = 128
LINE = 128


def _mf_kernel(b_per_w, n_chunks, n_feats, n_lines, tail_start, tail_pad):
    mesh = plsc.VectorSubcoreMesh(core_axis_name="c", subcore_axis_name="s")
    B = b_per_w * NW

    @functools.partial(
        pl.kernel,
        mesh=mesh,
        compiler_params=pltpu.CompilerParams(needs_layout_passes=False),
        out_type=jax.ShapeDtypeStruct((B,), jnp.float32),
        scratch_types=[
            pltpu.VMEM((CHUNK,), jnp.int32),
            pltpu.VMEM((CHUNK,), jnp.int32),
            pltpu.VMEM((CHUNK,), jnp.int32),
            pltpu.VMEM((CHUNK,), jnp.int32),
            pltpu.VMEM((CHUNK, LINE), jnp.float32),
            pltpu.VMEM((CHUNK, LINE), jnp.float32),
            pltpu.VMEM((tail_pad,), jnp.float32),
            pltpu.VMEM((tail_pad,), jnp.float32),
            pltpu.VMEM((n_feats, b_per_w), jnp.float32),
            pltpu.VMEM((n_feats, b_per_w), jnp.float32),
            pltpu.VMEM((b_per_w,), jnp.float32),
            pltpu.SemaphoreType.DMA,
        ],
    )
    def k(uraw_hbm, iraw_hbm, ubl_hbm, ibl_hbm, ubt_hbm, ibt_hbm,
          uvt_hbm, ivt_hbm, out_hbm,
          uraw_v, iraw_v, ul_v, il_v, ubs_v, ibs_v, ubt_v, ibt_v,
          us_v, is_v, out_v, sem):
        wid = lax.axis_index("s") * NC + lax.axis_index("c")
        base = wid * b_per_w
        wsl = pl.ds(base, b_per_w)

        rows_cp = [
            pltpu.async_copy(uvt_hbm.at[:, wsl], us_v, sem),
            pltpu.async_copy(ivt_hbm.at[:, wsl], is_v, sem),
        ]
        pltpu.sync_copy(ubt_hbm, ubt_v)
        pltpu.sync_copy(ibt_hbm, ibt_v)

        lane = lax.broadcasted_iota(jnp.int32, (L,), 0)

        for j in range(n_chunks):
            csl = pl.ds(base + j * CHUNK, CHUNK)
            pltpu.sync_copy(uraw_hbm.at[csl], uraw_v)
            pltpu.sync_copy(iraw_hbm.at[csl], iraw_v)

            def lines(g, _):
                gsl = pl.ds(g * L, L)
                ul_v[gsl] = jnp.minimum(uraw_v[gsl] >> 7, n_lines - 1)
                il_v[gsl] = jnp.minimum(iraw_v[gsl] >> 7, n_lines - 1)
                return _

            lax.fori_loop(0, CHUNK // L, lines, None)

            cps = [
                pltpu.async_copy(ubl_hbm.at[ul_v], ubs_v, sem),
                pltpu.async_copy(ibl_hbm.at[il_v], ibs_v, sem),
            ]
            if j == 0:
                cps = cps + rows_cp
            for c in cps:
                c.wait()

            def compute(g, _, _j=j):
                gsl = pl.ds(g * L, L)
                rows = g * L + lane
                ur = uraw_v[gsl]
                ir = iraw_v[gsl]
                ubv = plsc.load_gather(ubs_v, [rows, ur & (LINE - 1)])
                ibv = plsc.load_gather(ibs_v, [rows, ir & (LINE - 1)])
                ut = plsc.load_gather(
                    ubt_v, [jnp.clip(ur - tail_start, 0, tail_pad - 1)])
                it = plsc.load_gather(
                    ibt_v, [jnp.clip(ir - tail_start, 0, tail_pad - 1)])
                acc = (jnp.where(ur >= tail_start, ut, ubv)
                       + jnp.where(ir >= tail_start, it, ibv))
                bsl = pl.ds(_j * CHUNK + g * L, L)
                for f in range(n_feats):
                    acc = acc + us_v[f, bsl] * is_v[f, bsl]
                out_v[bsl] = acc
                return _

            lax.fori_loop(0, CHUNK // L, compute, None)

        pltpu.sync_copy(out_v, out_hbm.at[wsl])

    return k


def kernel(user, item, u_bias, i_bias, u_embed, i_embed):
    B = user.shape[0]
    n_rows, n_feats = u_embed.shape
    b_per_w = B // NW
    n_chunks = b_per_w // CHUNK

    user = user.astype(jnp.int32)
    item = item.astype(jnp.int32)

    uvt = jnp.take(u_embed, user, axis=0).T
    ivt = jnp.take(i_embed, item, axis=0).T

    n_lines = n_rows // LINE
    tail_start = n_lines * LINE
    tail_pad = -(-(n_rows - tail_start) // 8) * 8
    ubf = u_bias.reshape(-1)
    ibf = i_bias.reshape(-1)
    ubl = ubf[:tail_start].reshape(n_lines, LINE)
    ibl = ibf[:tail_start].reshape(n_lines, LINE)
    ubt = ubf[tail_start:]
    ibt = ibf[tail_start:]
    if tail_pad != ubt.shape[0]:
        ubt = jnp.pad(ubt, (0, tail_pad - ubt.shape[0]))
        ibt = jnp.pad(ibt, (0, tail_pad - ibt.shape[0]))

    k = _mf_kernel(b_per_w, n_chunks, n_feats, n_lines, tail_start, tail_pad)
    return k(user, item, ubl, ibl, ubt, ibt, uvt, ivt)


# speedup vs baseline: 5.5414x; 1.6371x over previous
import functools

import jax
import jax.numpy as jnp
from jax import lax
from jax.experimental import pallas as pl
from jax.experimental.pallas import tpu as pltpu
from jax.experimental.pallas import tpu_sc as plsc

NC = 2
NS = 16
NW = NC * NS
L = 16


def _mf_kernel(b_per_w, n_feats):
    mesh = plsc.VectorSubcoreMesh(core_axis_name="c", subcore_axis_name="s")
    B = b_per_w * NW

    @functools.partial(
        pl.kernel,
        mesh=mesh,
        compiler_params=pltpu.CompilerParams(needs_layout_passes=False),
        out_type=jax.ShapeDtypeStruct((B,), jnp.float32),
        scratch_types=[
            pltpu.VMEM((n_feats, b_per_w), jnp.float32),
            pltpu.VMEM((n_feats, b_per_w), jnp.float32),
            pltpu.VMEM((b_per_w,), jnp.float32),
            pltpu.VMEM((b_per_w,), jnp.float32),
            pltpu.VMEM((b_per_w,), jnp.float32),
            pltpu.SemaphoreType.DMA,
        ],
    )
    def k(uvt_hbm, ivt_hbm, ub_hbm, ib_hbm, out_hbm,
          us_v, is_v, ub_v, ib_v, out_v, sem):
        wid = lax.axis_index("s") * NC + lax.axis_index("c")
        base = wid * b_per_w
        wsl = pl.ds(base, b_per_w)

        cps = [
            pltpu.async_copy(uvt_hbm.at[:, wsl], us_v, sem),
            pltpu.async_copy(ivt_hbm.at[:, wsl], is_v, sem),
            pltpu.async_copy(ub_hbm.at[wsl], ub_v, sem),
            pltpu.async_copy(ib_hbm.at[wsl], ib_v, sem),
        ]
        for c in cps:
            c.wait()

        def compute(g, _):
            gsl = pl.ds(g * L, L)
            acc = ub_v[gsl] + ib_v[gsl]
            for f in range(n_feats):
                acc = acc + us_v[f, gsl] * is_v[f, gsl]
            out_v[gsl] = acc
            return _

        lax.fori_loop(0, b_per_w // L, compute, None)
        pltpu.sync_copy(out_v, out_hbm.at[wsl])

    return k


def kernel(user, item, u_bias, i_bias, u_embed, i_embed):
    B = user.shape[0]
    n_feats = u_embed.shape[1]
    b_per_w = B // NW

    uvt = jnp.take(u_embed, user, axis=0).T
    ivt = jnp.take(i_embed, item, axis=0).T
    ub = jnp.take(u_bias, user, axis=0).reshape(B)
    ib = jnp.take(i_bias, item, axis=0).reshape(B)

    k = _mf_kernel(b_per_w, n_feats)
    return k(uvt, ivt, ub, ib)


# speedup vs baseline: 5.5539x; 1.0023x over previous
import functools

import jax
import jax.numpy as jnp
from jax import lax
from jax.experimental import pallas as pl
from jax.experimental.pallas import tpu as pltpu
from jax.experimental.pallas import tpu_sc as plsc

NC = 2
NS = 16
NW = NC * NS
L = 16


def _mf_kernel(b_per_w, n_feats):
    mesh = plsc.VectorSubcoreMesh(core_axis_name="c", subcore_axis_name="s")
    B = b_per_w * NW

    @functools.partial(
        pl.kernel,
        mesh=mesh,
        compiler_params=pltpu.CompilerParams(needs_layout_passes=False),
        out_type=jax.ShapeDtypeStruct((B,), jnp.float32),
        scratch_types=[
            pltpu.VMEM((n_feats, b_per_w), jnp.float32),
            pltpu.VMEM((n_feats, b_per_w), jnp.float32),
            pltpu.VMEM((1, b_per_w), jnp.float32),
            pltpu.VMEM((1, b_per_w), jnp.float32),
            pltpu.VMEM((b_per_w,), jnp.float32),
            pltpu.SemaphoreType.DMA,
        ],
    )
    def k(uvt_hbm, ivt_hbm, ub_hbm, ib_hbm, out_hbm,
          us_v, is_v, ub_v, ib_v, out_v, sem):
        wid = lax.axis_index("s") * NC + lax.axis_index("c")
        base = wid * b_per_w
        wsl = pl.ds(base, b_per_w)

        cps = [
            pltpu.async_copy(uvt_hbm.at[:, wsl], us_v, sem),
            pltpu.async_copy(ivt_hbm.at[:, wsl], is_v, sem),
            pltpu.async_copy(ub_hbm.at[:, wsl], ub_v, sem),
            pltpu.async_copy(ib_hbm.at[:, wsl], ib_v, sem),
        ]
        for c in cps:
            c.wait()

        def compute(g, _):
            gsl = pl.ds(g * L, L)
            acc = ub_v[0, gsl] + ib_v[0, gsl]
            for f in range(n_feats):
                acc = acc + us_v[f, gsl] * is_v[f, gsl]
            out_v[gsl] = acc
            return _

        lax.fori_loop(0, b_per_w // L, compute, None)
        pltpu.sync_copy(out_v, out_hbm.at[wsl])

    return k


def kernel(user, item, u_bias, i_bias, u_embed, i_embed):
    B = user.shape[0]
    n_feats = u_embed.shape[1]
    b_per_w = B // NW

    uvt = jnp.take(u_embed, user, axis=0).T
    ivt = jnp.take(i_embed, item, axis=0).T
    ub = jnp.take(u_bias, user, axis=0).T
    ib = jnp.take(i_bias, item, axis=0).T

    k = _mf_kernel(b_per_w, n_feats)
    return k(uvt, ivt, ub, ib)
